# Initial kernel scaffold; baseline (speedup 1.0000x reference)
import jax
import jax.numpy as jnp
from jax.experimental import pallas as pl


def kernel(x, edge_index, edge_attr, W_inter, b_inter, W_pre1, b_pre1, cheb1_W, cheb1_b, cheb2_W, cheb2_b, W_post1, b_post1, W_post2, b_post2, W_out, b_out):
    raise NotImplementedError("write your pallas kernel here")


# speedup vs baseline: 3.8287x; 3.8287x over previous
import functools

import jax
import jax.numpy as jnp
from jax import lax
from jax.experimental import pallas as pl
from jax.experimental.pallas import tpu as pltpu
from jax.experimental.pallas import tpu_sc as plsc

N = 10000
NP = 10240
E = 320000
D = 128
DE = 16
AW = 32

NC = 2
NS = 16
NW = NC * NS
HN = NP // NC
ND = 16
K = 128

NCH = 157
EPT = NCH * K
EP = NS * EPT

NCHA = 79
EPW = NCHA * K
EA = NW * EPW

ZR = HN // NS
ZB = 64
ZRA = NP // NS

_mesh = plsc.VectorSubcoreMesh(core_axis_name="c", subcore_axis_name="s")

_F32 = jnp.float32


@functools.partial(
    pl.kernel,
    mesh=_mesh,
    out_type=jax.ShapeDtypeStruct((NP, D), _F32),
    scratch_types=[
        pltpu.VMEM((NCH, K), jnp.int32),
        pltpu.VMEM((NCH, K), jnp.int32),
        pltpu.VMEM((K, D), _F32),
        pltpu.VMEM_SHARED((HN + ND, D), _F32),
        pltpu.SemaphoreType.DMA,
    ],
)
def _diag_kernel(v_hbm, row_hbm, col_hbm, z_hbm, out_hbm,
                 ridx, cidx, bufa, acc, sema):
    c = lax.axis_index("c")
    s = lax.axis_index("s")
    pltpu.sync_copy(row_hbm.at[s], ridx)
    pltpu.sync_copy(col_hbm.at[c * NS + s], cidx)
    pltpu.sync_copy(z_hbm, bufa.at[pl.ds(0, ZB)])
    for q in range(ZR // ZB):
        pltpu.sync_copy(bufa.at[pl.ds(0, ZB)],
                        acc.at[pl.ds(s * ZR + q * ZB, ZB)])
    plsc.subcore_barrier()

    def dstep(i, carry):
        pltpu.async_copy(v_hbm.at[ridx.at[i]], bufa, sema).wait()
        pltpu.sync_copy(bufa, acc.at[cidx.at[i]], add=True)
        return carry

    lax.fori_loop(0, NCH, dstep, 0)
    plsc.subcore_barrier()
    for q in range(ZR // ZB):
        pltpu.sync_copy(acc.at[pl.ds(s * ZR + q * ZB, ZB)],
                        bufa.at[pl.ds(0, ZB)])
        pltpu.sync_copy(bufa.at[pl.ds(0, ZB)],
                        out_hbm.at[pl.ds(c * HN + s * ZR + q * ZB, ZB)])


R = 1024
G = NP // R

_HI = lax.Precision.HIGHEST


def _mm(a, b):
    return jnp.dot(a, b, precision=_HI, preferred_element_type=_F32)


def _tc1_body(s1, b1, b2, wa, wb, bi, wp, bp, h1o, u1o, dvo):
    s2 = b1[:, :DE]
    cnt = b1[:, DE:DE + 1]
    dg = b2[:, DE:DE + 1]
    num = _mm(s1[...], wa[...]) + _mm(s2, wb[...]) + cnt * bi[...]
    h = num / jnp.maximum(cnt, 1.0)
    h1 = jnp.maximum(_mm(h, wp[...]) + bp[...], 0.0)
    dinv = jnp.where(dg > 0.0, lax.rsqrt(dg), 0.0)
    h1o[...] = h1
    u1o[...] = dinv * h1
    dvo[...] = jnp.broadcast_to(dinv, h1.shape)


def _cheba_body(pp, dv, h, w0, w1, oo, uo):
    tx1 = -dv[...] * pp[...]
    oo[...] = _mm(h[...], w0[...]) + _mm(tx1, w1[...])
    uo[...] = dv[...] * tx1


def _chebb_body(pp, dv, h, op, w2, b, ho, uo):
    tx2 = -2.0 * dv[...] * pp[...] - h[...]
    hn = jnp.maximum(op[...] + _mm(tx2, w2[...]) + b[...], 0.0)
    ho[...] = hn
    uo[...] = dv[...] * hn


def _tail_body(pp, dv, h, op, w2, b2, wp1, bp1, wp2, bp2, wo, bo, out):
    tx2 = -2.0 * dv[...] * pp[...] - h[...]
    h3 = jnp.maximum(op[...] + _mm(tx2, w2[...]) + b2[...], 0.0)
    h4 = jnp.maximum(_mm(h3, wp1[...]) + bp1[...], 0.0)
    h5 = jnp.maximum(_mm(h4, wp2[...]) + bp2[...], 0.0)
    out[...] = _mm(h5, wo[...]) + bo[...]


def _full_spec(r, c):
    return pl.BlockSpec((r, c), lambda i: (0, 0))


def _row_spec(w):
    return pl.BlockSpec((R, w), lambda i: (i, 0))


def _sds(w):
    return jax.ShapeDtypeStruct((NP, w), _F32)


def _tc1(s1, b1, b2, wa, wb, bi, wp, bp):
    return pl.pallas_call(
        _tc1_body,
        grid=(G,),
        in_specs=[_row_spec(D), _row_spec(D), _row_spec(D),
                  _full_spec(D, D), _full_spec(DE, D), _full_spec(1, D),
                  _full_spec(D, D), _full_spec(1, D)],
        out_specs=[_row_spec(D), _row_spec(D), _row_spec(D)],
        out_shape=(_sds(D), _sds(D), _sds(D)),
    )(s1, b1, b2, wa, wb, bi, wp, bp)


def _cheba(pp, dv, h, w0, w1):
    return pl.pallas_call(
        _cheba_body,
        grid=(G,),
        in_specs=[_row_spec(D), _row_spec(D), _row_spec(D),
                  _full_spec(D, D), _full_spec(D, D)],
        out_specs=[_row_spec(D), _row_spec(D)],
        out_shape=(_sds(D), _sds(D)),
    )(pp, dv, h, w0, w1)


def _chebb(pp, dv, h, op, w2, b):
    return pl.pallas_call(
        _chebb_body,
        grid=(G,),
        in_specs=[_row_spec(D), _row_spec(D), _row_spec(D), _row_spec(D),
                  _full_spec(D, D), _full_spec(1, D)],
        out_specs=[_row_spec(D), _row_spec(D)],
        out_shape=(_sds(D), _sds(D)),
    )(pp, dv, h, op, w2, b)


def _tail(pp, dv, h, op, w2, b2, wp1, bp1, wp2, bp2, wo, bo):
    return pl.pallas_call(
        _tail_body,
        grid=(G,),
        in_specs=[_row_spec(D), _row_spec(D), _row_spec(D), _row_spec(D),
                  _full_spec(D, D), _full_spec(1, D),
                  _full_spec(D, D), _full_spec(1, D),
                  _full_spec(D, D), _full_spec(1, D),
                  _full_spec(D, 8), _full_spec(1, 8)],
        out_specs=[_row_spec(8)],
        out_shape=(_sds(8),),
    )(pp, dv, h, op, w2, b2, wp1, bp1, wp2, bp2, wo, bo)


def kernel(x, edge_index, edge_attr, W_inter, b_inter, W_pre1, b_pre1,
           cheb1_W, cheb1_b, cheb2_W, cheb2_b, W_post1, b_post1,
           W_post2, b_post2, W_out, b_out):
    row, col = edge_index[0], edge_index[1]

    padp = jnp.zeros((EP - E,), jnp.int32)
    rowp = jnp.concatenate([row, padp]).reshape(NS, NCH, K)
    dump = HN + (jnp.arange(E, dtype=jnp.int32) % ND)
    dump_pad = HN + (jnp.arange(EP - E, dtype=jnp.int32) % ND)
    colp = jnp.stack(
        [jnp.concatenate([jnp.where(col < HN, col, dump), dump_pad]),
         jnp.concatenate([jnp.where(col >= HN, col - HN, dump), dump_pad])],
        axis=0).reshape(NC * NS, NCH, K)

    rowh = jnp.stack(
        [jnp.concatenate([jnp.where(row < HN, row, dump), dump_pad]),
         jnp.concatenate([jnp.where(row >= HN, row - HN, dump), dump_pad])],
        axis=0).reshape(NC * NS, NCH, K)
    eaP = jnp.concatenate(
        [jnp.concatenate(
            [edge_attr.astype(_F32),
             jnp.ones((E, 1), _F32),
             jnp.zeros((E, D - DE - 1), _F32)], axis=1),
         jnp.zeros((EP - E, D), _F32)], axis=0)
    ids = jnp.arange(EP, dtype=jnp.int32).reshape(NS, NCH, K)
    zp = jnp.zeros((ZB, D), _F32)

    def P(v):
        return _diag_kernel(v, rowp, colp, zp)

    xp = jnp.pad(x, ((0, NP - N), (0, 0)))
    s1 = P(xp)
    s2cnt = _diag_kernel(eaP, ids, colp, zp)
    degv = _diag_kernel(eaP, ids, rowh, zp)

    h1, u1, dv = _tc1(s1, s2cnt, degv, W_inter[:D], W_inter[D:],
                      b_inter.reshape(1, D), W_pre1, b_pre1.reshape(1, D))

    o1, u2 = _cheba(P(u1), dv, h1, cheb1_W[0], cheb1_W[1])
    h2, u3 = _chebb(P(u2), dv, h1, o1, cheb1_W[2], cheb1_b.reshape(1, D))
    o2, u4 = _cheba(P(u3), dv, h2, cheb2_W[0], cheb2_W[1])

    wo8 = jnp.pad(W_out, ((0, 0), (0, 7)))
    bo8 = jnp.pad(b_out.reshape(1, 1), ((0, 0), (0, 7)))
    o8, = _tail(P(u4), dv, h2, o2, cheb2_W[2], cheb2_b.reshape(1, D),
                W_post1, b_post1.reshape(1, D), W_post2, b_post2.reshape(1, D),
                wo8, bo8)
    return o8[:N, 0]


# speedup vs baseline: 4.4117x; 1.1523x over previous
import functools

import jax
import jax.numpy as jnp
from jax import lax
from jax.experimental import pallas as pl
from jax.experimental.pallas import tpu as pltpu
from jax.experimental.pallas import tpu_sc as plsc

N = 10000
NP = 10240
E = 320000
D = 128
DE = 16
AW = 32

NC = 2
NS = 16
NW = NC * NS
HN = NP // NC
ND = 1024
K = 128

NCH = 157
EPT = NCH * K
EP = NS * EPT

NCHA = 79
EPW = NCHA * K
EA = NW * EPW

ZR = HN // NS
ZB = 64
ZRA = NP // NS

_mesh = plsc.VectorSubcoreMesh(core_axis_name="c", subcore_axis_name="s")

_F32 = jnp.float32


@functools.partial(
    pl.kernel,
    mesh=_mesh,
    out_type=jax.ShapeDtypeStruct((NP, D), _F32),
    scratch_types=[
        pltpu.VMEM((NCH, K), jnp.int32),
        pltpu.VMEM((NCH, K), jnp.int32),
        pltpu.VMEM((K, D), _F32),
        pltpu.VMEM((K, D), _F32),
        pltpu.VMEM_SHARED((HN + ND, D), _F32),
        pltpu.SemaphoreType.DMA,
        pltpu.SemaphoreType.DMA,
        pltpu.SemaphoreType.DMA,
        pltpu.SemaphoreType.DMA,
    ],
)
def _diag_kernel(v_hbm, row_hbm, col_hbm, z_hbm, out_hbm,
                 ridx, cidx, bufa, bufb, acc, sga, sgb, ssa, ssb):
    c = lax.axis_index("c")
    s = lax.axis_index("s")
    pltpu.sync_copy(row_hbm.at[s], ridx)
    pltpu.sync_copy(col_hbm.at[c * NS + s], cidx)
    pltpu.sync_copy(z_hbm, bufa.at[pl.ds(0, ZB)])
    for q in range(ZR // ZB):
        pltpu.sync_copy(bufa.at[pl.ds(0, ZB)],
                        acc.at[pl.ds(s * ZR + q * ZB, ZB)])
    plsc.subcore_barrier()

    def _waitg(i, buf, sem):
        pltpu.make_async_copy(v_hbm.at[ridx.at[i]], buf, sem).wait()

    def _waits(i, buf, sem):
        pltpu.make_async_copy(buf, acc.at[cidx.at[i]], sem).wait()

    pltpu.async_copy(v_hbm.at[ridx.at[0]], bufa, sga)
    pltpu.async_copy(v_hbm.at[ridx.at[1]], bufb, sgb)

    def dstep(j, carry):
        i = 2 * j
        _waitg(i, bufa, sga)
        pltpu.async_copy(bufa, acc.at[cidx.at[i]], ssa, add=True)
        _waitg(i + 1, bufb, sgb)
        pltpu.async_copy(bufb, acc.at[cidx.at[i + 1]], ssb, add=True)
        _waits(i, bufa, ssa)
        pltpu.async_copy(v_hbm.at[ridx.at[i + 2]], bufa, sga)
        _waits(i + 1, bufb, ssb)
        pltpu.async_copy(v_hbm.at[ridx.at[i + 3]], bufb, sgb)
        return carry

    lax.fori_loop(0, (NCH - 3) // 2, dstep, 0)
    _waitg(NCH - 3, bufa, sga)
    pltpu.async_copy(bufa, acc.at[cidx.at[NCH - 3]], ssa, add=True)
    _waitg(NCH - 2, bufb, sgb)
    pltpu.async_copy(bufb, acc.at[cidx.at[NCH - 2]], ssb, add=True)
    _waits(NCH - 3, bufa, ssa)
    pltpu.async_copy(v_hbm.at[ridx.at[NCH - 1]], bufa, sga)
    _waits(NCH - 2, bufb, ssb)
    _waitg(NCH - 1, bufa, sga)
    pltpu.async_copy(bufa, acc.at[cidx.at[NCH - 1]], ssa, add=True)
    _waits(NCH - 1, bufa, ssa)
    plsc.subcore_barrier()
    for q in range(ZR // ZB):
        pltpu.sync_copy(acc.at[pl.ds(s * ZR + q * ZB, ZB)],
                        bufa.at[pl.ds(0, ZB)])
        pltpu.sync_copy(bufa.at[pl.ds(0, ZB)],
                        out_hbm.at[pl.ds(c * HN + s * ZR + q * ZB, ZB)])


R = 1024
G = NP // R

_HI = lax.Precision.HIGHEST


def _mm(a, b):
    return jnp.dot(a, b, precision=_HI, preferred_element_type=_F32)


def _tc1_body(s1, b1, b2, wa, wb, bi, wp, bp, h1o, u1o, dvo):
    s2 = b1[:, :DE]
    cnt = b1[:, DE:DE + 1]
    dg = b2[:, DE:DE + 1]
    num = _mm(s1[...], wa[...]) + _mm(s2, wb[...]) + cnt * bi[...]
    h = num / jnp.maximum(cnt, 1.0)
    h1 = jnp.maximum(_mm(h, wp[...]) + bp[...], 0.0)
    dinv = jnp.where(dg > 0.0, lax.rsqrt(dg), 0.0)
    h1o[...] = h1
    u1o[...] = dinv * h1
    dvo[...] = jnp.broadcast_to(dinv, h1.shape)


def _cheba_body(pp, dv, h, w0, w1, oo, uo):
    tx1 = -dv[...] * pp[...]
    oo[...] = _mm(h[...], w0[...]) + _mm(tx1, w1[...])
    uo[...] = dv[...] * tx1


def _chebb_body(pp, dv, h, op, w2, b, ho, uo):
    tx2 = -2.0 * dv[...] * pp[...] - h[...]
    hn = jnp.maximum(op[...] + _mm(tx2, w2[...]) + b[...], 0.0)
    ho[...] = hn
    uo[...] = dv[...] * hn


def _tail_body(pp, dv, h, op, w2, b2, wp1, bp1, wp2, bp2, wo, bo, out):
    tx2 = -2.0 * dv[...] * pp[...] - h[...]
    h3 = jnp.maximum(op[...] + _mm(tx2, w2[...]) + b2[...], 0.0)
    h4 = jnp.maximum(_mm(h3, wp1[...]) + bp1[...], 0.0)
    h5 = jnp.maximum(_mm(h4, wp2[...]) + bp2[...], 0.0)
    out[...] = _mm(h5, wo[...]) + bo[...]


def _full_spec(r, c):
    return pl.BlockSpec((r, c), lambda i: (0, 0))


def _row_spec(w):
    return pl.BlockSpec((R, w), lambda i: (i, 0))


def _sds(w):
    return jax.ShapeDtypeStruct((NP, w), _F32)


def _tc1(s1, b1, b2, wa, wb, bi, wp, bp):
    return pl.pallas_call(
        _tc1_body,
        grid=(G,),
        in_specs=[_row_spec(D), _row_spec(D), _row_spec(D),
                  _full_spec(D, D), _full_spec(DE, D), _full_spec(1, D),
                  _full_spec(D, D), _full_spec(1, D)],
        out_specs=[_row_spec(D), _row_spec(D), _row_spec(D)],
        out_shape=(_sds(D), _sds(D), _sds(D)),
    )(s1, b1, b2, wa, wb, bi, wp, bp)


def _cheba(pp, dv, h, w0, w1):
    return pl.pallas_call(
        _cheba_body,
        grid=(G,),
        in_specs=[_row_spec(D), _row_spec(D), _row_spec(D),
                  _full_spec(D, D), _full_spec(D, D)],
        out_specs=[_row_spec(D), _row_spec(D)],
        out_shape=(_sds(D), _sds(D)),
    )(pp, dv, h, w0, w1)


def _chebb(pp, dv, h, op, w2, b):
    return pl.pallas_call(
        _chebb_body,
        grid=(G,),
        in_specs=[_row_spec(D), _row_spec(D), _row_spec(D), _row_spec(D),
                  _full_spec(D, D), _full_spec(1, D)],
        out_specs=[_row_spec(D), _row_spec(D)],
        out_shape=(_sds(D), _sds(D)),
    )(pp, dv, h, op, w2, b)


def _tail(pp, dv, h, op, w2, b2, wp1, bp1, wp2, bp2, wo, bo):
    return pl.pallas_call(
        _tail_body,
        grid=(G,),
        in_specs=[_row_spec(D), _row_spec(D), _row_spec(D), _row_spec(D),
                  _full_spec(D, D), _full_spec(1, D),
                  _full_spec(D, D), _full_spec(1, D),
                  _full_spec(D, D), _full_spec(1, D),
                  _full_spec(D, 8), _full_spec(1, 8)],
        out_specs=[_row_spec(8)],
        out_shape=(_sds(8),),
    )(pp, dv, h, op, w2, b2, wp1, bp1, wp2, bp2, wo, bo)


def kernel(x, edge_index, edge_attr, W_inter, b_inter, W_pre1, b_pre1,
           cheb1_W, cheb1_b, cheb2_W, cheb2_b, W_post1, b_post1,
           W_post2, b_post2, W_out, b_out):
    row, col = edge_index[0], edge_index[1]

    padp = jnp.zeros((EP - E,), jnp.int32)
    rowp = jnp.concatenate([row, padp]).reshape(NS, NCH, K)
    dump = HN + (jnp.arange(E, dtype=jnp.int32) % ND)
    dump_pad = HN + (jnp.arange(EP - E, dtype=jnp.int32) % ND)
    colp = jnp.stack(
        [jnp.concatenate([jnp.where(col < HN, col, dump), dump_pad]),
         jnp.concatenate([jnp.where(col >= HN, col - HN, dump), dump_pad])],
        axis=0).reshape(NC * NS, NCH, K)

    rowh = jnp.stack(
        [jnp.concatenate([jnp.where(row < HN, row, dump), dump_pad]),
         jnp.concatenate([jnp.where(row >= HN, row - HN, dump), dump_pad])],
        axis=0).reshape(NC * NS, NCH, K)
    eaP = jnp.concatenate(
        [jnp.concatenate(
            [edge_attr.astype(_F32),
             jnp.ones((E, 1), _F32),
             jnp.zeros((E, D - DE - 1), _F32)], axis=1),
         jnp.zeros((EP - E, D), _F32)], axis=0)
    ids = jnp.arange(EP, dtype=jnp.int32).reshape(NS, NCH, K)
    zp = jnp.zeros((ZB, D), _F32)

    def P(v):
        return _diag_kernel(v, rowp, colp, zp)

    xp = jnp.pad(x, ((0, NP - N), (0, 0)))
    s1 = P(xp)
    s2cnt = _diag_kernel(eaP, ids, colp, zp)
    degv = _diag_kernel(eaP, ids, rowh, zp)

    h1, u1, dv = _tc1(s1, s2cnt, degv, W_inter[:D], W_inter[D:],
                      b_inter.reshape(1, D), W_pre1, b_pre1.reshape(1, D))

    o1, u2 = _cheba(P(u1), dv, h1, cheb1_W[0], cheb1_W[1])
    h2, u3 = _chebb(P(u2), dv, h1, o1, cheb1_W[2], cheb1_b.reshape(1, D))
    o2, u4 = _cheba(P(u3), dv, h2, cheb2_W[0], cheb2_W[1])

    wo8 = jnp.pad(W_out, ((0, 0), (0, 7)))
    bo8 = jnp.pad(b_out.reshape(1, 1), ((0, 0), (0, 7)))
    o8, = _tail(P(u4), dv, h2, o2, cheb2_W[2], cheb2_b.reshape(1, D),
                W_post1, b_post1.reshape(1, D), W_post2, b_post2.reshape(1, D),
                wo8, bo8)
    return o8[:N, 0]
